# Initial kernel scaffold; baseline (speedup 1.0000x reference)
import jax
import jax.numpy as jnp
from jax.experimental import pallas as pl


def kernel(x):
    raise NotImplementedError("write your pallas kernel here")


# speedup vs baseline: 1.6338x; 1.6338x over previous
import dataclasses
import functools
import math

import jax
import jax.numpy as jnp
from jax import lax
from jax.experimental import pallas as pl
from jax.experimental.pallas import tpu as pltpu
from jax.experimental.pallas import tpu_sc as plsc

NBINS = 108
PF = 16
PT = 25
B = 8
F = 128
T = 500
TPAD = 512
NCHUNK = 64
ROWS = 16
CELLS_T = 20
HIST = CELLS_T * NBINS
CHUNK_ELEMS = ROWS * TPAD
NWORKERS = 32
CHUNKS_PER_W = NCHUNK // NWORKERS


def _tc_prep_body(x_ref, idx_ref, mag_ref):
    x = x_ref[...].astype(jnp.bfloat16).astype(jnp.float32)
    xp = jnp.concatenate([x[:, 1:2, :], x, x[:, F - 2:F - 1, :]], axis=1)
    xp = jnp.concatenate([xp[:, :, 1:2], xp, xp[:, :, T - 2:T - 1]], axis=2)
    t00 = xp[:, 0:F, 0:T]
    t02 = xp[:, 0:F, 2:T + 2]
    t10 = xp[:, 1:F + 1, 0:T]
    t12 = xp[:, 1:F + 1, 2:T + 2]
    t20 = xp[:, 2:F + 2, 0:T]
    t22 = xp[:, 2:F + 2, 2:T + 2]
    gx = t00 - t02 + 2.0 * t10 - 2.0 * t12 + t20 - t22
    gy = t00 - t20 + 2.0 * xp[:, 0:F, 1:T + 1] - 2.0 * xp[:, 2:F + 2, 1:T + 1] + t02 - t22
    mag = jnp.sqrt(gx * gx + gy * gy)
    phase = jnp.arctan2(gx, gy) / jnp.pi * NBINS
    bin_ = jnp.mod(jnp.floor(phase).astype(jnp.int32), NBINS)
    tb = lax.broadcasted_iota(jnp.int32, (B, F, T), 2) // PT
    idx = tb * NBINS + bin_
    zi = jnp.zeros((B, F, TPAD - T), jnp.int32)
    zm = jnp.zeros((B, F, TPAD - T), jnp.float32)
    idx_ref[...] = jnp.concatenate([idx, zi], axis=2)
    mag_ref[...] = jnp.concatenate([mag, zm], axis=2)


_tc_prep = pl.pallas_call(
    _tc_prep_body,
    out_shape=(
        jax.ShapeDtypeStruct((B, F, TPAD), jnp.int32),
        jax.ShapeDtypeStruct((B, F, TPAD), jnp.float32),
    ),
)


@functools.cache
def _make_sc_hist():
    mesh = plsc.VectorSubcoreMesh(core_axis_name="c", subcore_axis_name="s")
    cp = pltpu.CompilerParams()
    if "needs_layout_passes" in pltpu.CompilerParams.__dataclass_fields__:
        cp = dataclasses.replace(cp, needs_layout_passes=False)

    @functools.partial(
        pl.kernel,
        compiler_params=cp,
        out_type=jax.ShapeDtypeStruct((NCHUNK, HIST), jnp.float32),
        mesh=mesh,
        scratch_types=[
            pltpu.VMEM((CHUNK_ELEMS,), jnp.int32),
            pltpu.VMEM((CHUNK_ELEMS,), jnp.float32),
            pltpu.VMEM((HIST,), jnp.float32),
            pltpu.SemaphoreType.DMA,
        ],
    )
    def _sc_hist(idx_hbm, mag_hbm, out_hbm, idx_v, mag_v, hist_v, sem):
        wid = lax.axis_index("s") * 2 + lax.axis_index("c")
        for ci in range(CHUNKS_PER_W):
            chunk = wid * CHUNKS_PER_W + ci

            pltpu.async_copy(idx_hbm.at[chunk], idx_v, sem).wait()
            pltpu.async_copy(mag_hbm.at[chunk], mag_v, sem).wait()

            @pl.loop(0, HIST, step=16)
            def _zero(k):
                hist_v[pl.ds(k, 16)] = jnp.zeros((16,), jnp.float32)

            @pl.loop(0, CHUNK_ELEMS, step=16)
            def _scat(j):
                iv = idx_v[pl.ds(j, 16)]
                mv = mag_v[pl.ds(j, 16)]
                plsc.addupdate_scatter(hist_v, [iv], mv)

            pltpu.sync_copy(hist_v, out_hbm.at[chunk])

    return _sc_hist


def _tc_norm_body(h_ref, o_ref):
    h = h_ref[...]
    s = jnp.sum(h * h, axis=1, keepdims=True)
    nrm = jnp.maximum(jnp.sqrt(s), 1e-12)
    o_ref[...] = h / nrm


_tc_norm = pl.pallas_call(
    _tc_norm_body,
    out_shape=jax.ShapeDtypeStruct((NCHUNK * CELLS_T, NBINS), jnp.float32),
)


def _sobel_conv(x, w):
    return lax.conv_general_dilated(
        x, w, window_strides=(1, 1), padding='VALID',
        dimension_numbers=('NCHW', 'OIHW', 'NCHW'))


def kernel(x):
    sobel = jnp.array([[1., 0., -1.], [2., 0., -2.], [1., 0., -1.]],
                      dtype=jnp.float32)
    weight_t = sobel.reshape(1, 1, 3, 3)
    weight_f = sobel.T.reshape(1, 1, 3, 3)
    xp = jnp.pad(x, ((0, 0), (0, 0), (1, 1), (1, 1)), mode='reflect')
    gx = _sobel_conv(xp, weight_t)
    gy = _sobel_conv(xp, weight_f)
    grad_mag = jnp.sqrt(gx ** 2 + gy ** 2)
    phase = jnp.arctan2(gx, gy) / jnp.pi * NBINS
    phase_bin = jnp.floor(phase).astype(jnp.int32) % NBINS
    bins = phase_bin[:, 0]
    mag = grad_mag[:, 0]
    tb = (jnp.arange(T, dtype=jnp.int32) // PT)[None, None, :]
    idx = tb * NBINS + bins
    idxp = jnp.pad(idx, ((0, 0), (0, 0), (0, TPAD - T)))
    magp = jnp.pad(mag, ((0, 0), (0, 0), (0, TPAD - T)))
    hist = _make_sc_hist()(idxp.reshape(NCHUNK, CHUNK_ELEMS),
                           magp.reshape(NCHUNK, CHUNK_ELEMS))
    out = _tc_norm(hist.reshape(NCHUNK * CELLS_T, NBINS))
    return out.reshape(B, NCHUNK * CELLS_T // B, NBINS)


# speedup vs baseline: 2.1791x; 1.3338x over previous
import dataclasses
import functools
import math

import jax
import jax.numpy as jnp
from jax import lax
from jax.experimental import pallas as pl
from jax.experimental.pallas import tpu as pltpu
from jax.experimental.pallas import tpu_sc as plsc

NBINS = 108
PF = 16
PT = 25
B = 8
F = 128
T = 500
TPAD = 512
NCHUNK = 64
ROWS = 16
CELLS_T = 20
HIST = CELLS_T * NBINS
CHUNK_ELEMS = ROWS * TPAD
NWORKERS = 32
CHUNKS_PER_W = NCHUNK // NWORKERS


def _tc_prep_body(x_ref, idx_ref, mag_ref):
    x = x_ref[...].astype(jnp.bfloat16).astype(jnp.float32)
    xp = jnp.concatenate([x[:, 1:2, :], x, x[:, F - 2:F - 1, :]], axis=1)
    xp = jnp.concatenate([xp[:, :, 1:2], xp, xp[:, :, T - 2:T - 1]], axis=2)
    t00 = xp[:, 0:F, 0:T]
    t02 = xp[:, 0:F, 2:T + 2]
    t10 = xp[:, 1:F + 1, 0:T]
    t12 = xp[:, 1:F + 1, 2:T + 2]
    t20 = xp[:, 2:F + 2, 0:T]
    t22 = xp[:, 2:F + 2, 2:T + 2]
    gx = t00 - t02 + 2.0 * t10 - 2.0 * t12 + t20 - t22
    gy = t00 - t20 + 2.0 * xp[:, 0:F, 1:T + 1] - 2.0 * xp[:, 2:F + 2, 1:T + 1] + t02 - t22
    mag = jnp.sqrt(gx * gx + gy * gy)
    phase = jnp.arctan2(gx, gy) / jnp.pi * NBINS
    bin_ = jnp.mod(jnp.floor(phase).astype(jnp.int32), NBINS)
    tb = lax.broadcasted_iota(jnp.int32, (B, F, T), 2) // PT
    idx = tb * NBINS + bin_
    zi = jnp.zeros((B, F, TPAD - T), jnp.int32)
    zm = jnp.zeros((B, F, TPAD - T), jnp.float32)
    idx_ref[...] = jnp.concatenate([idx, zi], axis=2)
    mag_ref[...] = jnp.concatenate([mag, zm], axis=2)


_tc_prep = pl.pallas_call(
    _tc_prep_body,
    out_shape=(
        jax.ShapeDtypeStruct((B, F, TPAD), jnp.int32),
        jax.ShapeDtypeStruct((B, F, TPAD), jnp.float32),
    ),
)


@functools.cache
def _make_sc_hist():
    mesh = plsc.VectorSubcoreMesh(core_axis_name="c", subcore_axis_name="s")
    cp = pltpu.CompilerParams()
    if "needs_layout_passes" in pltpu.CompilerParams.__dataclass_fields__:
        cp = dataclasses.replace(cp, needs_layout_passes=False)

    @functools.partial(
        pl.kernel,
        compiler_params=cp,
        out_type=jax.ShapeDtypeStruct((NCHUNK, HIST), jnp.float32),
        mesh=mesh,
        scratch_types=[
            pltpu.VMEM((CHUNK_ELEMS,), jnp.int32),
            pltpu.VMEM((CHUNK_ELEMS,), jnp.float32),
            pltpu.VMEM((HIST,), jnp.float32),
            pltpu.SemaphoreType.DMA,
        ],
    )
    def _sc_hist(idx_hbm, mag_hbm, out_hbm, idx_v, mag_v, hist_v, sem):
        wid = lax.axis_index("s") * 2 + lax.axis_index("c")
        for ci in range(CHUNKS_PER_W):
            chunk = wid * CHUNKS_PER_W + ci

            pltpu.async_copy(idx_hbm.at[chunk], idx_v, sem).wait()
            pltpu.async_copy(mag_hbm.at[chunk], mag_v, sem).wait()

            @pl.loop(0, HIST, step=16)
            def _zero(k):
                hist_v[pl.ds(k, 16)] = jnp.zeros((16,), jnp.float32)

            @pl.loop(0, CHUNK_ELEMS, step=16)
            def _scat(j):
                iv = idx_v[pl.ds(j, 16)]
                mv = mag_v[pl.ds(j, 16)]
                plsc.addupdate_scatter(hist_v, [iv], mv)

            pltpu.sync_copy(hist_v, out_hbm.at[chunk])

    return _sc_hist


def _tc_norm_body(h_ref, o_ref):
    h = h_ref[...]
    s = jnp.sum(h * h, axis=1, keepdims=True)
    nrm = jnp.maximum(jnp.sqrt(s), 1e-12)
    o_ref[...] = h / nrm


_tc_norm = pl.pallas_call(
    _tc_norm_body,
    out_shape=jax.ShapeDtypeStruct((NCHUNK * CELLS_T, NBINS), jnp.float32),
)


def _tc_hist_body(bins_ref, mag_ref, o_ref, h_ref):
    bins = bins_ref[...]
    mag = mag_ref[...]
    ti = lax.broadcasted_iota(jnp.int32, (T, 128), 0)
    ci = lax.broadcasted_iota(jnp.int32, (T, 128), 1)
    pt = (ti // PT == ci).astype(jnp.float32)

    def body(bb, carry):
        m = jnp.where(bins == bb, mag, 0.0)
        mf = m.reshape(B, B, PF, T).sum(axis=2)
        h_ref[bb] = lax.dot_general(mf.reshape(NCHUNK, T), pt,
                                    (((1,), (0,)), ((), ())),
                                    preferred_element_type=jnp.float32)
        return carry

    lax.fori_loop(0, NBINS, body, 0)
    h = h_ref[...]
    s = jnp.sum(h * h, axis=0)
    nrm = jnp.maximum(jnp.sqrt(s), 1e-12)
    o_ref[...] = (h / nrm[None])[:, :, :CELLS_T]


_tc_hist = pl.pallas_call(
    _tc_hist_body,
    out_shape=jax.ShapeDtypeStruct((NBINS, NCHUNK, CELLS_T), jnp.float32),
    scratch_shapes=[pltpu.VMEM((NBINS, NCHUNK, 128), jnp.float32)],
)


def _sobel_conv(x, w):
    return lax.conv_general_dilated(
        x, w, window_strides=(1, 1), padding='VALID',
        dimension_numbers=('NCHW', 'OIHW', 'NCHW'))


def kernel(x):
    sobel = jnp.array([[1., 0., -1.], [2., 0., -2.], [1., 0., -1.]],
                      dtype=jnp.float32)
    weight_t = sobel.reshape(1, 1, 3, 3)
    weight_f = sobel.T.reshape(1, 1, 3, 3)
    xp = jnp.pad(x, ((0, 0), (0, 0), (1, 1), (1, 1)), mode='reflect')
    gx = _sobel_conv(xp, weight_t)
    gy = _sobel_conv(xp, weight_f)
    grad_mag = jnp.sqrt(gx ** 2 + gy ** 2)
    phase = jnp.arctan2(gx, gy) / jnp.pi * NBINS
    phase_bin = jnp.floor(phase).astype(jnp.int32) % NBINS
    bins = phase_bin[:, 0]
    mag = grad_mag[:, 0]
    h = _tc_hist(bins, mag)
    return jnp.transpose(h, (1, 2, 0)).reshape(B, NCHUNK * CELLS_T // B, NBINS)
